# baseline (speedup 1.0000x reference)
import functools

import jax
import jax.numpy as jnp
from jax import lax
from jax.experimental import pallas as pl
from jax.experimental.pallas import tpu as pltpu
from jax.experimental.pallas import tpu_sc as plsc

NPTS = 262144
NC, NS = 2, 16
NW = NC * NS
PPW = NPTS // NW
CH = 32
NCHUNK = PPW // CH
DEPTH = 8
SLICE = NPTS // NS
IMG_N = 512

_mesh = plsc.VectorSubcoreMesh(core_axis_name="c", subcore_axis_name="s")


@functools.partial(
    pl.kernel,
    mesh=_mesh,
    out_type=jax.ShapeDtypeStruct((NC, 2, NPTS), jnp.float32),
    scratch_types=[
        pltpu.VMEM((PPW,), jnp.float32),
        pltpu.VMEM((PPW,), jnp.float32),
        pltpu.VMEM((DEPTH, 4 * CH), jnp.int32),
        pltpu.VMEM((DEPTH, 4 * CH), jnp.float32),
        pltpu.VMEM((DEPTH, 4 * CH), jnp.float32),
        pltpu.VMEM((SLICE,), jnp.float32),
        pltpu.VMEM_SHARED((NPTS,), jnp.float32),
        pltpu.VMEM_SHARED((NPTS,), jnp.float32),
        pltpu.SemaphoreType.DMA,
    ],
)
def _sc_fields(zt_hbm, zero_hbm, out_hbm,
               z0_v, z1_v, idx_v, vx_v, vy_v, buf_v, a0_sh, a1_sh, sem):
    c = lax.axis_index("c")
    s = lax.axis_index("s")
    wid = s * NC + c

    pltpu.sync_copy(zero_hbm, buf_v)
    pltpu.sync_copy(buf_v, a0_sh.at[pl.ds(s * SLICE, SLICE)])
    pltpu.sync_copy(buf_v, a1_sh.at[pl.ds(s * SLICE, SLICE)])
    plsc.subcore_barrier()

    pltpu.sync_copy(zt_hbm.at[0, pl.ds(wid * PPW, PPW)], z0_v)
    pltpu.sync_copy(zt_hbm.at[1, pl.ds(wid * PPW, PPW)], z1_v)

    def build_chunk(k, d):
        for j in range(CH // 16):
            off = k * CH + j * 16
            yf = z0_v[pl.ds(off, 16)] * float(IMG_N - 1)
            xf = z1_v[pl.ds(off, 16)] * float(IMG_N - 1)
            y = yf.astype(jnp.int32)
            x = xf.astype(jnp.int32)
            fx = x.astype(jnp.float32) - xf
            fy = y.astype(jnp.float32) - yf
            p = y * IMG_N + x
            one = jnp.float32(1.0)
            for g, (dp, wx, wy) in enumerate((
                    (0, -one - fx, -one - fy),
                    (1, fx, one + fy),
                    (IMG_N, one + fx, fy),
                    (IMG_N + 1, -fx, -fy),
            )):
                r = g * CH + j * 16
                idx_v[d, pl.ds(r, 16)] = p + dp
                vx_v[d, pl.ds(r, 16)] = wx
                vy_v[d, pl.ds(r, 16)] = wy

    def fire(d):
        h0 = pltpu.async_copy(vx_v.at[d], a0_sh.at[idx_v.at[d]], sem, add=True)
        h1 = pltpu.async_copy(vy_v.at[d], a1_sh.at[idx_v.at[d]], sem, add=True)
        return h0, h1

    def round_(kk, _):
        handles = []
        for d in range(DEPTH):
            build_chunk(kk * DEPTH + d, d)
            handles.append(fire(d))
        for h0, h1 in handles:
            h0.wait()
            h1.wait()
        return 0

    lax.fori_loop(0, NCHUNK // DEPTH, round_, 0)

    plsc.subcore_barrier()
    pltpu.sync_copy(a0_sh.at[pl.ds(s * SLICE, SLICE)], buf_v)
    pltpu.sync_copy(buf_v, out_hbm.at[c, 0, pl.ds(s * SLICE, SLICE)])
    pltpu.sync_copy(a1_sh.at[pl.ds(s * SLICE, SLICE)], buf_v)
    pltpu.sync_copy(buf_v, out_hbm.at[c, 1, pl.ds(s * SLICE, SLICE)])


_ZR = 4096


def _tc_split_body(zr_ref, e_ref, o_ref, zt_ref):
    blk = zr_ref[...]
    hi = jax.lax.Precision.HIGHEST
    zt_ref[0, :, :] = jax.lax.dot(blk, e_ref[...], precision=hi)
    zt_ref[1, :, :] = jax.lax.dot(blk, o_ref[...], precision=hi)


_tc_split = pl.pallas_call(
    _tc_split_body,
    out_shape=jax.ShapeDtypeStruct((2, _ZR, 64), jnp.float32),
)


_KB = 4096
_GRID = NPTS // _KB


def _tc_body(img_ref, a_ref, out_ref):
    k = pl.program_id(0)
    blk = img_ref[...]
    a = a_ref[...]
    a0 = a[0, 0, :] + a[1, 0, :]
    a1 = a[0, 1, :] + a[1, 1, :]
    p0 = jnp.sum(blk * a0[None, :], axis=1, keepdims=True)
    p1 = jnp.sum(blk * a1[None, :], axis=1, keepdims=True)
    part = jnp.concatenate([p0, p1], axis=1)

    @pl.when(k == 0)
    def _():
        out_ref[...] = part

    @pl.when(k > 0)
    def _():
        out_ref[...] += part


_tc_contract = pl.pallas_call(
    _tc_body,
    grid=(_GRID,),
    in_specs=[
        pl.BlockSpec((24, _KB), lambda k: (0, k)),
        pl.BlockSpec((2, 2, _KB), lambda k: (0, 0, k)),
    ],
    out_specs=pl.BlockSpec((24, 2), lambda k: (0, 0)),
    out_shape=jax.ShapeDtypeStruct((24, 2), jnp.float32),
)


def kernel(img, z):
    zeros_tile = jnp.zeros((SLICE,), jnp.float32)
    lanes = jnp.arange(128, dtype=jnp.int32)[:, None]
    cols2 = 2 * jnp.arange(64, dtype=jnp.int32)[None, :]
    e_sel = (lanes == cols2).astype(jnp.float32)
    o_sel = (lanes == cols2 + 1).astype(jnp.float32)
    zt = _tc_split(z.reshape(_ZR, 128), e_sel, o_sel).reshape(2, NPTS)
    a2 = _sc_fields(zt, zeros_tile)
    img2 = img.reshape(24, NPTS)
    out24 = _tc_contract(img2, a2)
    return out24.reshape(8, 3, 2).sum(axis=1)


# speedup vs baseline: 1.0314x; 1.0314x over previous
import functools

import jax
import jax.numpy as jnp
from jax import lax
from jax.experimental import pallas as pl
from jax.experimental.pallas import tpu as pltpu
from jax.experimental.pallas import tpu_sc as plsc

NPTS = 262144
NC, NS = 2, 16
NW = NC * NS
PPW = NPTS // NW
CH = 32
NCHUNK = PPW // CH
DEPTH = 8
SLICE = NPTS // NS
IMG_N = 512

_mesh = plsc.VectorSubcoreMesh(core_axis_name="c", subcore_axis_name="s")


@functools.partial(
    pl.kernel,
    mesh=_mesh,
    out_type=jax.ShapeDtypeStruct((NC, 2, NPTS), jnp.float32),
    scratch_types=[
        pltpu.VMEM((PPW // 64, 128), jnp.float32),
        pltpu.VMEM((DEPTH, 4 * CH), jnp.int32),
        pltpu.VMEM((DEPTH, 4 * CH), jnp.float32),
        pltpu.VMEM((DEPTH, 4 * CH), jnp.float32),
        pltpu.VMEM((SLICE,), jnp.float32),
        pltpu.VMEM_SHARED((NPTS,), jnp.float32),
        pltpu.VMEM_SHARED((NPTS,), jnp.float32),
        pltpu.SemaphoreType.DMA,
    ],
)
def _sc_fields(zg_hbm, zero_hbm, out_hbm,
               z_v, idx_v, vx_v, vy_v, buf_v, a0_sh, a1_sh, sem):
    c = lax.axis_index("c")
    s = lax.axis_index("s")
    wid = s * NC + c

    pltpu.sync_copy(zero_hbm, buf_v)
    pltpu.sync_copy(buf_v, a0_sh.at[pl.ds(s * SLICE, SLICE)])
    pltpu.sync_copy(buf_v, a1_sh.at[pl.ds(s * SLICE, SLICE)])
    plsc.subcore_barrier()

    pltpu.sync_copy(zg_hbm.at[pl.ds(wid * (PPW // 64), PPW // 64)], z_v)

    def build_chunk(k, d):
        for j in range(CH // 16):
            off = k * CH + j * 16
            r_loc = off // 64
            c_loc = off % 64
            yf = z_v[r_loc, pl.ds(c_loc, 16)] * float(IMG_N - 1)
            xf = z_v[r_loc, pl.ds(c_loc + 64, 16)] * float(IMG_N - 1)
            y = yf.astype(jnp.int32)
            x = xf.astype(jnp.int32)
            fx = x.astype(jnp.float32) - xf
            fy = y.astype(jnp.float32) - yf
            p = y * IMG_N + x
            one = jnp.float32(1.0)
            for g, (dp, wx, wy) in enumerate((
                    (0, -one - fx, -one - fy),
                    (1, fx, one + fy),
                    (IMG_N, one + fx, fy),
                    (IMG_N + 1, -fx, -fy),
            )):
                r = g * CH + j * 16
                idx_v[d, pl.ds(r, 16)] = p + dp
                vx_v[d, pl.ds(r, 16)] = wx
                vy_v[d, pl.ds(r, 16)] = wy

    def fire(d):
        h0 = pltpu.async_copy(vx_v.at[d], a0_sh.at[idx_v.at[d]], sem, add=True)
        h1 = pltpu.async_copy(vy_v.at[d], a1_sh.at[idx_v.at[d]], sem, add=True)
        return h0, h1

    def round_(kk, _):
        handles = []
        for d in range(DEPTH):
            build_chunk(kk * DEPTH + d, d)
            handles.append(fire(d))
        for h0, h1 in handles:
            h0.wait()
            h1.wait()
        return 0

    lax.fori_loop(0, NCHUNK // DEPTH, round_, 0)

    plsc.subcore_barrier()
    pltpu.sync_copy(a0_sh.at[pl.ds(s * SLICE, SLICE)], buf_v)
    pltpu.sync_copy(buf_v, out_hbm.at[c, 0, pl.ds(s * SLICE, SLICE)])
    pltpu.sync_copy(a1_sh.at[pl.ds(s * SLICE, SLICE)], buf_v)
    pltpu.sync_copy(buf_v, out_hbm.at[c, 1, pl.ds(s * SLICE, SLICE)])


_ZR = 4096


def _tc_split_body(zr_ref, eo_ref, zt_ref):
    blk = zr_ref[...]
    hi = jax.lax.Precision.HIGHEST
    zt_ref[...] = jax.lax.dot(blk, eo_ref[...], precision=hi)


_ZRB = 1024

_tc_split = pl.pallas_call(
    _tc_split_body,
    grid=(_ZR // _ZRB,),
    in_specs=[
        pl.BlockSpec((_ZRB, 128), lambda k: (k, 0)),
        pl.BlockSpec((128, 128), lambda k: (0, 0)),
    ],
    out_specs=pl.BlockSpec((_ZRB, 128), lambda k: (k, 0)),
    out_shape=jax.ShapeDtypeStruct((_ZR, 128), jnp.float32),
)


_KB = 4096
_GRID = NPTS // _KB


def _tc_body(img_ref, a_ref, out_ref):
    k = pl.program_id(0)
    blk = img_ref[...]
    a = a_ref[...]
    a0 = a[0, 0, :] + a[1, 0, :]
    a1 = a[0, 1, :] + a[1, 1, :]
    p0 = jnp.sum(blk * a0[None, :], axis=1, keepdims=True)
    p1 = jnp.sum(blk * a1[None, :], axis=1, keepdims=True)
    part = jnp.concatenate([p0, p1], axis=1)

    @pl.when(k == 0)
    def _():
        out_ref[...] = part

    @pl.when(k > 0)
    def _():
        out_ref[...] += part


_tc_contract = pl.pallas_call(
    _tc_body,
    grid=(_GRID,),
    in_specs=[
        pl.BlockSpec((24, _KB), lambda k: (0, k)),
        pl.BlockSpec((2, 2, _KB), lambda k: (0, 0, k)),
    ],
    out_specs=pl.BlockSpec((24, 2), lambda k: (0, 0)),
    out_shape=jax.ShapeDtypeStruct((24, 2), jnp.float32),
)


def kernel(img, z):
    zeros_tile = jnp.zeros((SLICE,), jnp.float32)
    lanes = jnp.arange(128, dtype=jnp.int32)[:, None]
    cols2 = 2 * jnp.arange(64, dtype=jnp.int32)[None, :]
    eo_sel = jnp.concatenate(
        [lanes == cols2, lanes == cols2 + 1], axis=1).astype(jnp.float32)
    zg = _tc_split(z.reshape(_ZR, 128), eo_sel)
    a2 = _sc_fields(zg, zeros_tile)
    img2 = img.reshape(24, NPTS)
    out24 = _tc_contract(img2, a2)
    return out24.reshape(8, 3, 2).sum(axis=1)


# speedup vs baseline: 1.1822x; 1.1462x over previous
import functools

import jax
import jax.numpy as jnp
from jax import lax
from jax.experimental import pallas as pl
from jax.experimental.pallas import tpu as pltpu
from jax.experimental.pallas import tpu_sc as plsc

NPTS = 262144
NC, NS = 2, 16
NW = NC * NS
PPW = NPTS // NW
CH = 32
NCHUNK = PPW // CH
DEPTH = 8
SLICE = NPTS // NS
IMG_N = 512

_mesh = plsc.VectorSubcoreMesh(core_axis_name="c", subcore_axis_name="s")


@functools.partial(
    pl.kernel,
    mesh=_mesh,
    out_type=jax.ShapeDtypeStruct((NC, 2, NPTS), jnp.float32),
    scratch_types=[
        pltpu.VMEM((PPW // 64, 128), jnp.float32),
        pltpu.VMEM((DEPTH, 4 * CH), jnp.int32),
        pltpu.VMEM((DEPTH, 4 * CH), jnp.float32),
        pltpu.VMEM((DEPTH, 4 * CH), jnp.float32),
        pltpu.VMEM((SLICE,), jnp.float32),
        pltpu.VMEM_SHARED((NPTS,), jnp.float32),
        pltpu.VMEM_SHARED((NPTS,), jnp.float32),
        pltpu.SemaphoreType.DMA,
    ],
)
def _sc_fields(zg_hbm, zero_hbm, out_hbm,
               z_v, idx_v, vx_v, vy_v, buf_v, a0_sh, a1_sh, sem):
    c = lax.axis_index("c")
    s = lax.axis_index("s")
    wid = s * NC + c

    pltpu.sync_copy(zero_hbm, buf_v)
    pltpu.sync_copy(buf_v, a0_sh.at[pl.ds(s * SLICE, SLICE)])
    pltpu.sync_copy(buf_v, a1_sh.at[pl.ds(s * SLICE, SLICE)])
    plsc.subcore_barrier()

    pltpu.sync_copy(zg_hbm.at[pl.ds(wid * (PPW // 64), PPW // 64)], z_v)

    def build_chunk(k, d):
        for j in range(CH // 16):
            off = k * CH + j * 16
            r_loc = off // 64
            c_loc = off % 64
            yf = z_v[r_loc, pl.ds(c_loc, 16)] * float(IMG_N - 1)
            xf = z_v[r_loc, pl.ds(c_loc + 64, 16)] * float(IMG_N - 1)
            y = yf.astype(jnp.int32)
            x = xf.astype(jnp.int32)
            fx = x.astype(jnp.float32) - xf
            fy = y.astype(jnp.float32) - yf
            p = y * IMG_N + x
            one = jnp.float32(1.0)
            for g, (dp, wx, wy) in enumerate((
                    (0, -one - fx, -one - fy),
                    (1, fx, one + fy),
                    (IMG_N, one + fx, fy),
                    (IMG_N + 1, -fx, -fy),
            )):
                r = g * CH + j * 16
                idx_v[d, pl.ds(r, 16)] = p + dp
                vx_v[d, pl.ds(r, 16)] = wx
                vy_v[d, pl.ds(r, 16)] = wy

    def fire(d):
        h0 = pltpu.async_copy(vx_v.at[d], a0_sh.at[idx_v.at[d]], sem, add=True)
        h1 = pltpu.async_copy(vy_v.at[d], a1_sh.at[idx_v.at[d]], sem, add=True)
        return h0, h1

    def round_(kk, _):
        handles = []
        for d in range(DEPTH):
            build_chunk(kk * DEPTH + d, d)
            handles.append(fire(d))
        for h0, h1 in handles:
            h0.wait()
            h1.wait()
        return 0

    lax.fori_loop(0, NCHUNK // DEPTH, round_, 0)

    plsc.subcore_barrier()
    pltpu.sync_copy(a0_sh.at[pl.ds(s * SLICE, SLICE)], buf_v)
    pltpu.sync_copy(buf_v, out_hbm.at[c, 0, pl.ds(s * SLICE, SLICE)])
    pltpu.sync_copy(a1_sh.at[pl.ds(s * SLICE, SLICE)], buf_v)
    pltpu.sync_copy(buf_v, out_hbm.at[c, 1, pl.ds(s * SLICE, SLICE)])


_ZR = 4096


def _tc_split_body(zr_ref, eo_ref, zt_ref):
    blk = zr_ref[...]
    hi = jax.lax.Precision.HIGHEST
    zt_ref[...] = jax.lax.dot(blk, eo_ref[...], precision=hi)


_ZRB = 1024

_tc_split = pl.pallas_call(
    _tc_split_body,
    grid=(_ZR // _ZRB,),
    in_specs=[
        pl.BlockSpec((_ZRB, 128), lambda k: (k, 0)),
        pl.BlockSpec((128, 128), lambda k: (0, 0)),
    ],
    out_specs=pl.BlockSpec((_ZRB, 128), lambda k: (k, 0)),
    out_shape=jax.ShapeDtypeStruct((_ZR, 128), jnp.float32),
)


def _tc_body(img_ref, a_ref, out_ref):
    blk = img_ref[...]
    s2 = blk[0, 0] + blk[0, 1] + blk[0, 2]
    a = a_ref[...]
    a0 = a[0, 0] + a[1, 0]
    a1 = a[0, 1] + a[1, 1]
    b = pl.program_id(0)
    p0 = jnp.sum(s2 * a0).reshape(1, 1)
    p1 = jnp.sum(s2 * a1).reshape(1, 1)
    out_ref[pl.ds(b, 1), :] = jnp.concatenate([p0, p1], axis=1)


_tc_contract = pl.pallas_call(
    _tc_body,
    grid=(8,),
    in_specs=[
        pl.BlockSpec((1, 3, IMG_N, IMG_N), lambda b: (b, 0, 0, 0)),
        pl.BlockSpec((2, 2, IMG_N, IMG_N), lambda b: (0, 0, 0, 0)),
    ],
    out_specs=pl.BlockSpec((8, 2), lambda b: (0, 0)),
    out_shape=jax.ShapeDtypeStruct((8, 2), jnp.float32),
)


def kernel(img, z):
    zeros_tile = jnp.zeros((SLICE,), jnp.float32)
    lanes = jnp.arange(128, dtype=jnp.int32)[:, None]
    cols2 = 2 * jnp.arange(64, dtype=jnp.int32)[None, :]
    eo_sel = jnp.concatenate(
        [lanes == cols2, lanes == cols2 + 1], axis=1).astype(jnp.float32)
    zg = _tc_split(z.reshape(_ZR, 128), eo_sel)
    a2 = _sc_fields(zg, zeros_tile)
    a4 = a2.reshape(2, 2, IMG_N, IMG_N)
    return _tc_contract(img, a4)


# speedup vs baseline: 3.4038x; 2.8792x over previous
import functools

import jax
import jax.numpy as jnp
from jax import lax
from jax.experimental import pallas as pl
from jax.experimental.pallas import tpu as pltpu
from jax.experimental.pallas import tpu_sc as plsc

NPTS = 262144
NC, NS = 2, 16
NW = NC * NS
PPW = NPTS // NW
CH = 32
NCHUNK = PPW // CH
DEPTH = 8
SLICE = NPTS // NS
IMG_N = 512

_mesh = plsc.VectorSubcoreMesh(core_axis_name="c", subcore_axis_name="s")


@functools.partial(
    pl.kernel,
    mesh=_mesh,
    out_type=jax.ShapeDtypeStruct((NC, 2, NPTS), jnp.float32),
    scratch_types=[
        pltpu.VMEM((PPW,), jnp.float32),
        pltpu.VMEM((PPW,), jnp.float32),
        pltpu.VMEM((DEPTH, 4 * CH), jnp.int32),
        pltpu.VMEM((DEPTH, 4 * CH), jnp.float32),
        pltpu.VMEM((DEPTH, 4 * CH), jnp.float32),
        pltpu.VMEM((SLICE,), jnp.float32),
        pltpu.VMEM_SHARED((NPTS,), jnp.float32),
        pltpu.VMEM_SHARED((NPTS,), jnp.float32),
        pltpu.SemaphoreType.DMA,
    ],
)
def _sc_fields(z0_hbm, z1_hbm, zero_hbm, out_hbm,
               z0_v, z1_v, idx_v, vx_v, vy_v, buf_v, a0_sh, a1_sh, sem):
    c = lax.axis_index("c")
    s = lax.axis_index("s")
    wid = s * NC + c

    pltpu.sync_copy(zero_hbm, buf_v)
    pltpu.sync_copy(buf_v, a0_sh.at[pl.ds(s * SLICE, SLICE)])
    pltpu.sync_copy(buf_v, a1_sh.at[pl.ds(s * SLICE, SLICE)])
    plsc.subcore_barrier()

    pltpu.sync_copy(z0_hbm.at[pl.ds(wid * PPW, PPW)], z0_v)
    pltpu.sync_copy(z1_hbm.at[pl.ds(wid * PPW, PPW)], z1_v)

    def build_chunk(k, d):
        for j in range(CH // 16):
            off = k * CH + j * 16
            yf = z0_v[pl.ds(off, 16)] * float(IMG_N - 1)
            xf = z1_v[pl.ds(off, 16)] * float(IMG_N - 1)
            y = yf.astype(jnp.int32)
            x = xf.astype(jnp.int32)
            fx = x.astype(jnp.float32) - xf
            fy = y.astype(jnp.float32) - yf
            p = y * IMG_N + x
            one = jnp.float32(1.0)
            for g, (dp, wx, wy) in enumerate((
                    (0, -one - fx, -one - fy),
                    (1, fx, one + fy),
                    (IMG_N, one + fx, fy),
                    (IMG_N + 1, -fx, -fy),
            )):
                r = g * CH + j * 16
                idx_v[d, pl.ds(r, 16)] = p + dp
                vx_v[d, pl.ds(r, 16)] = wx
                vy_v[d, pl.ds(r, 16)] = wy

    def fire(d):
        h0 = pltpu.async_copy(vx_v.at[d], a0_sh.at[idx_v.at[d]], sem, add=True)
        h1 = pltpu.async_copy(vy_v.at[d], a1_sh.at[idx_v.at[d]], sem, add=True)
        return h0, h1

    def round_(kk, _):
        handles = []
        for d in range(DEPTH):
            build_chunk(kk * DEPTH + d, d)
            handles.append(fire(d))
        for h0, h1 in handles:
            h0.wait()
            h1.wait()
        return 0

    lax.fori_loop(0, NCHUNK // DEPTH, round_, 0)

    plsc.subcore_barrier()
    pltpu.sync_copy(a0_sh.at[pl.ds(s * SLICE, SLICE)], buf_v)
    pltpu.sync_copy(buf_v, out_hbm.at[c, 0, pl.ds(s * SLICE, SLICE)])
    pltpu.sync_copy(a1_sh.at[pl.ds(s * SLICE, SLICE)], buf_v)
    pltpu.sync_copy(buf_v, out_hbm.at[c, 1, pl.ds(s * SLICE, SLICE)])


def _tc_body(img_ref, a_ref, out_ref):
    blk = img_ref[...]
    s2 = blk[0, 0] + blk[0, 1] + blk[0, 2]
    a = a_ref[...]
    a0 = a[0, 0] + a[1, 0]
    a1 = a[0, 1] + a[1, 1]
    b = pl.program_id(0)
    p0 = jnp.sum(s2 * a0).reshape(1, 1)
    p1 = jnp.sum(s2 * a1).reshape(1, 1)
    out_ref[pl.ds(b, 1), :] = jnp.concatenate([p0, p1], axis=1)


_tc_contract = pl.pallas_call(
    _tc_body,
    grid=(8,),
    in_specs=[
        pl.BlockSpec((1, 3, IMG_N, IMG_N), lambda b: (b, 0, 0, 0)),
        pl.BlockSpec((2, 2, IMG_N, IMG_N), lambda b: (0, 0, 0, 0)),
    ],
    out_specs=pl.BlockSpec((8, 2), lambda b: (0, 0)),
    out_shape=jax.ShapeDtypeStruct((8, 2), jnp.float32),
)


def kernel(img, z):
    zeros_tile = jnp.zeros((SLICE,), jnp.float32)
    a2 = _sc_fields(z[:, 0], z[:, 1], zeros_tile)
    a4 = a2.reshape(2, 2, IMG_N, IMG_N)
    return _tc_contract(img, a4)
